# baseline (device time: 120153 ns/iter reference)
import jax
import jax.numpy as jnp
from jax import lax
from jax.experimental import pallas as pl
from jax.experimental.pallas import tpu as pltpu

N_Y = 4


def kernel(x, assign, W1, W2):
    t, d = x.shape
    n_local, _, f = W1.shape

    af = assign.astype(jnp.float32).reshape(t, 1)

    def body(x_ref, a_ref, w1_ref, w2_ref, out_ref,
             xg, ag, acc, part,
             p1sx, p1sa, p1rx, p1ra, p2s, p2r):
        my_x = lax.axis_index("x")
        my_y = lax.axis_index("y")
        my_z = lax.axis_index("z")

        barrier = pltpu.get_barrier_semaphore()
        for k in range(1, N_Y):
            pl.semaphore_signal(
                barrier, inc=1,
                device_id=(my_x, (my_y + k) % N_Y, my_z),
                device_id_type=pl.DeviceIdType.MESH,
            )
        pl.semaphore_wait(barrier, N_Y - 1)

        p1 = []
        for k in range(1, N_Y):
            dst = (my_x, (my_y + k) % N_Y, my_z)
            rx = pltpu.make_async_remote_copy(
                src_ref=x_ref, dst_ref=xg.at[k - 1],
                send_sem=p1sx.at[k - 1], recv_sem=p1rx.at[k - 1],
                device_id=dst, device_id_type=pl.DeviceIdType.MESH,
            )
            ra = pltpu.make_async_remote_copy(
                src_ref=a_ref, dst_ref=ag.at[k - 1],
                send_sem=p1sa.at[k - 1], recv_sem=p1ra.at[k - 1],
                device_id=dst, device_id_type=pl.DeviceIdType.MESH,
            )
            rx.start()
            ra.start()
            p1.append((rx, ra))

        e0 = (2 * my_y).astype(jnp.float32)

        def moe_partial(xb, ab):
            h0 = jnp.maximum(
                jnp.dot(xb, w1_ref[0], preferred_element_type=jnp.float32), 0.0)
            y0 = jnp.dot(h0, w2_ref[0], preferred_element_type=jnp.float32)
            h1 = jnp.maximum(
                jnp.dot(xb, w1_ref[1], preferred_element_type=jnp.float32), 0.0)
            y1 = jnp.dot(h1, w2_ref[1], preferred_element_type=jnp.float32)
            return jnp.where(ab == e0, y0, 0.0) + jnp.where(ab == e0 + 1.0, y1, 0.0)

        out_ref[:, :] = moe_partial(x_ref[:, :], a_ref[:, :])

        p2 = []
        for s in range(N_Y - 1):
            rx, ra = p1[s]
            rx.wait_recv()
            ra.wait_recv()
            part[s, :, :] = moe_partial(xg[s], ag[s])
            o = (my_x, (my_y - (s + 1)) % N_Y, my_z)
            r2 = pltpu.make_async_remote_copy(
                src_ref=part.at[s], dst_ref=acc.at[s],
                send_sem=p2s.at[s], recv_sem=p2r.at[s],
                device_id=o, device_id_type=pl.DeviceIdType.MESH,
            )
            r2.start()
            p2.append(r2)

        for s in range(N_Y - 1):
            p2[s].wait_recv()
            out_ref[:, :] = out_ref[:, :] + acc[s]

        for s in range(N_Y - 1):
            rx, ra = p1[s]
            rx.wait_send()
            ra.wait_send()
            p2[s].wait_send()

    return pl.pallas_call(
        body,
        out_shape=jax.ShapeDtypeStruct((t, d), jnp.float32),
        in_specs=[
            pl.BlockSpec(memory_space=pltpu.VMEM),
            pl.BlockSpec(memory_space=pltpu.VMEM),
            pl.BlockSpec(memory_space=pltpu.VMEM),
            pl.BlockSpec(memory_space=pltpu.VMEM),
        ],
        out_specs=pl.BlockSpec(memory_space=pltpu.VMEM),
        scratch_shapes=[
            pltpu.VMEM((N_Y - 1, t, d), jnp.float32),
            pltpu.VMEM((N_Y - 1, t, 1), jnp.float32),
            pltpu.VMEM((N_Y - 1, t, d), jnp.float32),
            pltpu.VMEM((N_Y - 1, t, d), jnp.float32),
            pltpu.SemaphoreType.DMA((N_Y - 1,)),
            pltpu.SemaphoreType.DMA((N_Y - 1,)),
            pltpu.SemaphoreType.DMA((N_Y - 1,)),
            pltpu.SemaphoreType.DMA((N_Y - 1,)),
            pltpu.SemaphoreType.DMA((N_Y - 1,)),
            pltpu.SemaphoreType.DMA((N_Y - 1,)),
        ],
        compiler_params=pltpu.CompilerParams(collective_id=0),
    )(x, af, W1, W2)


# device time: 34475 ns/iter; 3.4852x vs baseline; 3.4852x over previous
import jax
import jax.numpy as jnp
from jax import lax
from jax.experimental import pallas as pl
from jax.experimental.pallas import tpu as pltpu

N_Y = 4
N_R = 8
TS = 64
CAPS = 32


def kernel(x, assign, W1, W2):
    t, d = x.shape
    my_x = lax.axis_index("x")
    my_y = lax.axis_index("y")
    my_z = lax.axis_index("z")
    r8 = my_x * N_Y + my_z

    a_sub = lax.dynamic_slice_in_dim(assign, r8 * TS, TS, 0)
    x_sub = lax.dynamic_slice_in_dim(x, r8 * TS, TS, 0)
    slot = ((a_sub // 2) - my_y) % N_Y
    slot_col = slot.astype(jnp.float32).reshape(TS, 1)
    slot_row = slot.astype(jnp.float32).reshape(1, TS)
    e1_col = (a_sub % 2 + 1).astype(jnp.float32).reshape(TS, 1)

    def body(x_ref, sc_ref, sr_ref, e1_ref, w1_ref, w2_ref, out_ref,
             xsend, csend, xg, cg, acc, part, pmat, xzbar,
             p1sx, p1sc, p1rx, p1rc, p2s, p2r, ags, agrs):
        f32 = jnp.float32
        my_x = lax.axis_index("x")
        my_y = lax.axis_index("y")
        my_z = lax.axis_index("z")
        r8 = my_x * N_Y + my_z

        barrier = pltpu.get_barrier_semaphore()
        for k in range(1, N_Y):
            pl.semaphore_signal(
                barrier, inc=1,
                device_id=(my_x, (my_y + k) % N_Y, my_z),
                device_id_type=pl.DeviceIdType.MESH,
            )
        for q in range(1, N_R):
            rp = (r8 + q) % N_R
            pl.semaphore_signal(
                xzbar, inc=1,
                device_id=(rp // N_Y, my_y, rp % N_Y),
                device_id_type=pl.DeviceIdType.MESH,
            )
        pl.semaphore_wait(barrier, N_Y - 1)

        io4t_0 = lax.broadcasted_iota(jnp.int32, (N_Y, TS), 0).astype(f32)
        h_row = jnp.where(sr_ref[:, :] == io4t_0, 1.0, 0.0)
        upper = jnp.where(
            lax.broadcasted_iota(jnp.int32, (TS, TS), 0)
            <= lax.broadcasted_iota(jnp.int32, (TS, TS), 1),
            1.0, 0.0)
        cum_row = jnp.dot(h_row, upper, preferred_element_type=f32)
        rank_row = jnp.sum(h_row * cum_row, axis=0, keepdims=True) - 1.0

        iota_cap0 = lax.broadcasted_iota(jnp.int32, (CAPS, TS), 0).astype(f32)

        def p_gather(k):
            return jnp.where(
                (sr_ref[:, :] == float(k)) & (rank_row == iota_cap0), 1.0, 0.0)

        for k in range(1, N_Y):
            pk = p_gather(k)
            xsend[k - 1, :, :] = jnp.dot(pk, x_ref[:, :],
                                         preferred_element_type=f32)
            csend[k - 1, :, :] = jnp.dot(pk, e1_ref[:, :],
                                         preferred_element_type=f32) - 1.0

        p1 = []
        for k in range(1, N_Y):
            dst_id = (my_x, (my_y + k) % N_Y, my_z)
            rx = pltpu.make_async_remote_copy(
                src_ref=xsend.at[k - 1], dst_ref=xg.at[k - 1],
                send_sem=p1sx.at[k - 1], recv_sem=p1rx.at[k - 1],
                device_id=dst_id, device_id_type=pl.DeviceIdType.MESH,
            )
            rc = pltpu.make_async_remote_copy(
                src_ref=csend.at[k - 1], dst_ref=cg.at[k - 1],
                send_sem=p1sc.at[k - 1], recv_sem=p1rc.at[k - 1],
                device_id=dst_id, device_id_type=pl.DeviceIdType.MESH,
            )
            rx.start()
            rc.start()
            p1.append((rx, rc))

        io4t_1 = lax.broadcasted_iota(jnp.int32, (TS, N_Y), 1).astype(f32)
        h_col = jnp.where(sc_ref[:, :] == io4t_1, 1.0, 0.0)
        lower = jnp.where(
            lax.broadcasted_iota(jnp.int32, (TS, TS), 1)
            <= lax.broadcasted_iota(jnp.int32, (TS, TS), 0),
            1.0, 0.0)
        cum_col = jnp.dot(lower, h_col, preferred_element_type=f32)
        rank_col = jnp.sum(h_col * cum_col, axis=1, keepdims=True) - 1.0
        iota_cap1 = lax.broadcasted_iota(jnp.int32, (TS, CAPS), 1).astype(f32)
        for k in range(N_Y):
            pmat[k, :, :] = jnp.where(
                (sc_ref[:, :] == float(k)) & (rank_col == iota_cap1), 1.0, 0.0)

        def moe_group(xb, cb):
            h0 = jnp.maximum(
                jnp.dot(xb, w1_ref[0], preferred_element_type=f32), 0.0)
            y0 = jnp.dot(h0, w2_ref[0], preferred_element_type=f32)
            h1 = jnp.maximum(
                jnp.dot(xb, w1_ref[1], preferred_element_type=f32), 0.0)
            y1 = jnp.dot(h1, w2_ref[1], preferred_element_type=f32)
            return jnp.where(cb == 0.0, y0, 0.0) + jnp.where(cb == 1.0, y1, 0.0)

        p0 = p_gather(0)
        xs0 = jnp.dot(p0, x_ref[:, :], preferred_element_type=f32)
        c0 = jnp.dot(p0, e1_ref[:, :], preferred_element_type=f32) - 1.0
        sub = jnp.dot(pmat[0], moe_group(xs0, c0), preferred_element_type=f32)

        p2 = []
        for s in range(N_Y - 1):
            rx, rc = p1[s]
            rx.wait_recv()
            rc.wait_recv()
            part[s, :, :] = moe_group(xg[s], cg[s])
            o_id = (my_x, (my_y - (s + 1)) % N_Y, my_z)
            r2 = pltpu.make_async_remote_copy(
                src_ref=part.at[s], dst_ref=acc.at[s],
                send_sem=p2s.at[s], recv_sem=p2r.at[s],
                device_id=o_id, device_id_type=pl.DeviceIdType.MESH,
            )
            r2.start()
            p2.append(r2)

        for s in range(N_Y - 1):
            p2[s].wait_recv()
            sub = sub + jnp.dot(pmat[s + 1], acc[s], preferred_element_type=f32)
        out_ref[pl.ds(r8 * TS, TS), :] = sub

        pl.semaphore_wait(xzbar, N_R - 1)
        ag = []
        for q in range(1, N_R):
            rp = (r8 + q) % N_R
            r3 = pltpu.make_async_remote_copy(
                src_ref=out_ref.at[pl.ds(r8 * TS, TS), :],
                dst_ref=out_ref.at[pl.ds(r8 * TS, TS), :],
                send_sem=ags.at[q - 1], recv_sem=agrs.at[q - 1],
                device_id=(rp // N_Y, my_y, rp % N_Y),
                device_id_type=pl.DeviceIdType.MESH,
            )
            r3.start()
            ag.append(r3)
        for q in range(1, N_R):
            ag[q - 1].wait_recv()

        for s in range(N_Y - 1):
            rx, rc = p1[s]
            rx.wait_send()
            rc.wait_send()
            p2[s].wait_send()
        for q in range(1, N_R):
            ag[q - 1].wait_send()

    return pl.pallas_call(
        body,
        out_shape=jax.ShapeDtypeStruct((t, d), jnp.float32),
        in_specs=[pl.BlockSpec(memory_space=pltpu.VMEM)] * 6,
        out_specs=pl.BlockSpec(memory_space=pltpu.VMEM),
        scratch_shapes=[
            pltpu.VMEM((N_Y - 1, CAPS, d), jnp.float32),
            pltpu.VMEM((N_Y - 1, CAPS, 1), jnp.float32),
            pltpu.VMEM((N_Y - 1, CAPS, d), jnp.float32),
            pltpu.VMEM((N_Y - 1, CAPS, 1), jnp.float32),
            pltpu.VMEM((N_Y - 1, CAPS, d), jnp.float32),
            pltpu.VMEM((N_Y - 1, CAPS, d), jnp.float32),
            pltpu.VMEM((N_Y, TS, CAPS), jnp.float32),
            pltpu.SemaphoreType.REGULAR,
            pltpu.SemaphoreType.DMA((N_Y - 1,)),
            pltpu.SemaphoreType.DMA((N_Y - 1,)),
            pltpu.SemaphoreType.DMA((N_Y - 1,)),
            pltpu.SemaphoreType.DMA((N_Y - 1,)),
            pltpu.SemaphoreType.DMA((N_Y - 1,)),
            pltpu.SemaphoreType.DMA((N_Y - 1,)),
            pltpu.SemaphoreType.DMA((N_R - 1,)),
            pltpu.SemaphoreType.DMA((N_R - 1,)),
        ],
        compiler_params=pltpu.CompilerParams(collective_id=0),
    )(x_sub, slot_col, slot_row, e1_col, W1, W2)
